# baseline (speedup 1.0000x reference)
import jax
import jax.numpy as jnp
from jax.experimental import pallas as pl

_UV = 10000
_DIN = 300
_HID = 64
_DOUT = 64
_B = 4096

_T = 400
_G = 256


def _z1_body(embed_ref, w1_ref, b1_ref, o_ref):
    o_ref[...] = (
        jnp.dot(embed_ref[...], w1_ref[...], preferred_element_type=jnp.float32)
        + b1_ref[...]
    )


def _z2_body(adj_ref, z1_ref, w2_ref, o_ref):
    a = adj_ref[...].astype(jnp.float32)
    h = jnp.dot(a, z1_ref[...], preferred_element_type=jnp.float32)
    h = jnp.maximum(h, 0.0)
    o_ref[...] = jnp.dot(h, w2_ref[...], preferred_element_type=jnp.float32)


def _y_body(adj_ref, z2_ref, b2_ref, o_ref):
    a = adj_ref[...].astype(jnp.float32)
    o_ref[...] = (
        jnp.dot(a, z2_ref[...], preferred_element_type=jnp.float32) + b2_ref[...]
    )


def _gather_body(tid_ref, y_ref, o_ref):
    tid = tid_ref[0]
    iota = jax.lax.broadcasted_iota(jnp.int32, (_UV, _G), 0)
    oht = (iota == tid).astype(jnp.float32)
    o_ref[...] = jax.lax.dot_general(
        oht, y_ref[...], (((0,), (0,)), ((), ())),
        preferred_element_type=jnp.float32,
    )


def kernel(X_tid, adj, embed, W1, b1, W2, b2):
    b1r = jnp.reshape(b1, (1, _HID))
    b2r = jnp.reshape(b2, (1, _DOUT))

    z1b = pl.pallas_call(
        _z1_body,
        out_shape=jax.ShapeDtypeStruct((_UV, _HID), jnp.float32),
        in_specs=[
            pl.BlockSpec((_UV, _DIN), lambda: (0, 0)),
            pl.BlockSpec((_DIN, _HID), lambda: (0, 0)),
            pl.BlockSpec((1, _HID), lambda: (0, 0)),
        ],
        out_specs=pl.BlockSpec((_UV, _HID), lambda: (0, 0)),
    )(embed, W1, b1r)

    nt = _UV // _T
    z2 = pl.pallas_call(
        _z2_body,
        grid=(nt,),
        out_shape=jax.ShapeDtypeStruct((_UV, _DOUT), jnp.float32),
        in_specs=[
            pl.BlockSpec((_T, _UV), lambda i: (i, 0)),
            pl.BlockSpec((_UV, _HID), lambda i: (0, 0)),
            pl.BlockSpec((_HID, _DOUT), lambda i: (0, 0)),
        ],
        out_specs=pl.BlockSpec((_T, _DOUT), lambda i: (i, 0)),
    )(adj, z1b, W2)

    y = pl.pallas_call(
        _y_body,
        grid=(nt,),
        out_shape=jax.ShapeDtypeStruct((_UV, _DOUT), jnp.float32),
        in_specs=[
            pl.BlockSpec((_T, _UV), lambda i: (i, 0)),
            pl.BlockSpec((_UV, _DOUT), lambda i: (0, 0)),
            pl.BlockSpec((1, _DOUT), lambda i: (0, 0)),
        ],
        out_specs=pl.BlockSpec((_T, _DOUT), lambda i: (i, 0)),
    )(adj, z2, b2r)

    ng = _B // _G
    tid3 = jnp.reshape(X_tid, (ng, 1, _G))
    out = pl.pallas_call(
        _gather_body,
        grid=(ng,),
        out_shape=jax.ShapeDtypeStruct((_B, _DOUT), jnp.float32),
        in_specs=[
            pl.BlockSpec((1, 1, _G), lambda i: (i, 0, 0)),
            pl.BlockSpec((_UV, _DOUT), lambda i: (0, 0)),
        ],
        out_specs=pl.BlockSpec((_G, _DOUT), lambda i: (i, 0)),
    )(tid3, y)
    return out


# speedup vs baseline: 1.2568x; 1.2568x over previous
import jax
import jax.numpy as jnp
from jax.experimental import pallas as pl
from jax.experimental.pallas import tpu as pltpu

_UV = 10000
_DIN = 300
_HID = 64
_DOUT = 64
_B = 4096

_T = 400
_G = 128


def _z1_body(embed_ref, w1_ref, b1_ref, o_ref):
    o_ref[...] = (
        jnp.dot(embed_ref[...], w1_ref[...], preferred_element_type=jnp.float32)
        + b1_ref[...]
    )


def _z2_body(adj_ref, z1_ref, w2_ref, o_ref):
    a = adj_ref[...].astype(jnp.float32)
    h = jnp.dot(a, z1_ref[...], preferred_element_type=jnp.float32)
    h = jnp.maximum(h, 0.0)
    o_ref[...] = jnp.dot(h, w2_ref[...], preferred_element_type=jnp.float32)


def _out_body(tid_ref, adj_ref, z2_ref, b2_ref, o_ref, buf, sems):
    i = pl.program_id(0)
    ng = _B // _G

    def _copy(slot, g, b):
        row = tid_ref[b]
        return pltpu.make_async_copy(
            adj_ref.at[pl.ds(row, 1), :],
            buf.at[slot, pl.ds(g, 1), :],
            sems.at[slot],
        )

    def _issue(slot, grp):
        def one(g, _):
            _copy(slot, g, grp * _G + g).start()
            return 0
        jax.lax.fori_loop(0, _G, one, 0)

    @pl.when(i == 0)
    def _():
        _issue(0, 0)

    @pl.when(i + 1 < ng)
    def _():
        _issue((i + 1) % 2, i + 1)

    slot = i % 2

    def waitone(g, _):
        _copy(slot, g, i * _G + g).wait()
        return 0
    jax.lax.fori_loop(0, _G, waitone, 0)

    a = buf[slot].astype(jnp.float32)
    o_ref[...] = (
        jnp.dot(a, z2_ref[...], preferred_element_type=jnp.float32) + b2_ref[...]
    )


def kernel(X_tid, adj, embed, W1, b1, W2, b2):
    b1r = jnp.reshape(b1, (1, _HID))
    b2r = jnp.reshape(b2, (1, _DOUT))

    z1b = pl.pallas_call(
        _z1_body,
        out_shape=jax.ShapeDtypeStruct((_UV, _HID), jnp.float32),
        in_specs=[
            pl.BlockSpec((_UV, _DIN), lambda: (0, 0)),
            pl.BlockSpec((_DIN, _HID), lambda: (0, 0)),
            pl.BlockSpec((1, _HID), lambda: (0, 0)),
        ],
        out_specs=pl.BlockSpec((_UV, _HID), lambda: (0, 0)),
    )(embed, W1, b1r)

    nt = _UV // _T
    z2 = pl.pallas_call(
        _z2_body,
        grid=(nt,),
        out_shape=jax.ShapeDtypeStruct((_UV, _DOUT), jnp.float32),
        in_specs=[
            pl.BlockSpec((_T, _UV), lambda i: (i, 0)),
            pl.BlockSpec((_UV, _HID), lambda i: (0, 0)),
            pl.BlockSpec((_HID, _DOUT), lambda i: (0, 0)),
        ],
        out_specs=pl.BlockSpec((_T, _DOUT), lambda i: (i, 0)),
    )(adj, z1b, W2)

    ng = _B // _G
    out = pl.pallas_call(
        _out_body,
        grid_spec=pltpu.PrefetchScalarGridSpec(
            num_scalar_prefetch=1,
            grid=(ng,),
            in_specs=[
                pl.BlockSpec(memory_space=pl.ANY),
                pl.BlockSpec((_UV, _DOUT), lambda i, tid: (0, 0)),
                pl.BlockSpec((1, _DOUT), lambda i, tid: (0, 0)),
            ],
            out_specs=pl.BlockSpec((_G, _DOUT), lambda i, tid: (i, 0)),
            scratch_shapes=[
                pltpu.VMEM((2, _G, _UV), jnp.int32),
                pltpu.SemaphoreType.DMA((2,)),
            ],
        ),
        out_shape=jax.ShapeDtypeStruct((_B, _DOUT), jnp.float32),
    )(X_tid, adj, z2, b2r)
    return out


# speedup vs baseline: 1.5034x; 1.1962x over previous
import jax
import jax.numpy as jnp
from jax.experimental import pallas as pl
from jax.experimental.pallas import tpu as pltpu

_UV = 10000
_DIN = 300
_HID = 64
_DOUT = 64
_B = 4096

_T = 400
_G = 256


def _z1_body(embed_ref, w1_ref, b1_ref, o_ref):
    o_ref[...] = (
        jnp.dot(embed_ref[...], w1_ref[...], preferred_element_type=jnp.float32)
        + b1_ref[...]
    )


def _z2_body(adj_ref, z1_ref, w2_ref, o_ref):
    a = adj_ref[...].astype(jnp.float32)
    h = jnp.dot(a, z1_ref[...], preferred_element_type=jnp.float32)
    h = jnp.maximum(h, 0.0)
    o_ref[...] = jnp.dot(h, w2_ref[...], preferred_element_type=jnp.float32)


def _out_body(tid_ref, adj_ref, z2_ref, b2_ref, o_ref, buf, sems):
    i = pl.program_id(0)
    ng = _B // _G

    def _issue(slot, grp):
        for g in range(_G):
            pltpu.make_async_copy(
                adj_ref.at[pl.ds(tid_ref[grp * _G + g], 1), :],
                buf.at[slot, pl.ds(g, 1), :],
                sems.at[slot],
            ).start()

    @pl.when(i == 0)
    def _():
        _issue(0, 0)

    @pl.when(i + 1 < ng)
    def _():
        _issue((i + 1) % 2, i + 1)

    slot = i % 2
    pltpu.make_async_copy(
        adj_ref.at[pl.ds(0, _G), :], buf.at[slot], sems.at[slot]
    ).wait()

    a = buf[slot].astype(jnp.float32)
    o_ref[...] = (
        jnp.dot(a, z2_ref[...], preferred_element_type=jnp.float32) + b2_ref[...]
    )


def kernel(X_tid, adj, embed, W1, b1, W2, b2):
    b1r = jnp.reshape(b1, (1, _HID))
    b2r = jnp.reshape(b2, (1, _DOUT))

    z1b = pl.pallas_call(
        _z1_body,
        out_shape=jax.ShapeDtypeStruct((_UV, _HID), jnp.float32),
        in_specs=[
            pl.BlockSpec((_UV, _DIN), lambda: (0, 0)),
            pl.BlockSpec((_DIN, _HID), lambda: (0, 0)),
            pl.BlockSpec((1, _HID), lambda: (0, 0)),
        ],
        out_specs=pl.BlockSpec((_UV, _HID), lambda: (0, 0)),
    )(embed, W1, b1r)

    nt = _UV // _T
    z2 = pl.pallas_call(
        _z2_body,
        grid=(nt,),
        out_shape=jax.ShapeDtypeStruct((_UV, _DOUT), jnp.float32),
        in_specs=[
            pl.BlockSpec((_T, _UV), lambda i: (i, 0)),
            pl.BlockSpec((_UV, _HID), lambda i: (0, 0)),
            pl.BlockSpec((_HID, _DOUT), lambda i: (0, 0)),
        ],
        out_specs=pl.BlockSpec((_T, _DOUT), lambda i: (i, 0)),
    )(adj, z1b, W2)

    ng = _B // _G
    out = pl.pallas_call(
        _out_body,
        grid_spec=pltpu.PrefetchScalarGridSpec(
            num_scalar_prefetch=1,
            grid=(ng,),
            in_specs=[
                pl.BlockSpec(memory_space=pl.ANY),
                pl.BlockSpec((_UV, _DOUT), lambda i, tid: (0, 0)),
                pl.BlockSpec((1, _DOUT), lambda i, tid: (0, 0)),
            ],
            out_specs=pl.BlockSpec((_G, _DOUT), lambda i, tid: (i, 0)),
            scratch_shapes=[
                pltpu.VMEM((2, _G, _UV), jnp.int32),
                pltpu.SemaphoreType.DMA((2,)),
            ],
        ),
        out_shape=jax.ShapeDtypeStruct((_B, _DOUT), jnp.float32),
    )(X_tid, adj, z2, b2r)
    return out
